# baseline (speedup 1.0000x reference)
import functools

import jax
import jax.numpy as jnp
from jax import lax
from jax.experimental import pallas as pl
from jax.experimental.pallas import tpu as pltpu
from jax.experimental.pallas import tpu_sc as plsc

D_MODEL = 64
SCALE = 8.0
CHUNK = 128
LANES = 16


@functools.partial(jax.jit, static_argnames=("n_chunks", "n_workers"))
def _embed_sc(x1d, table, *, n_chunks, n_workers):
    b_total = x1d.shape[0]
    bpw = n_chunks * CHUNK
    info = plsc.get_sparse_core_info()
    nc, ns = info.num_cores, info.num_subcores
    assert nc * ns == n_workers
    mesh = plsc.VectorSubcoreMesh(core_axis_name="c", subcore_axis_name="s")

    @functools.partial(
        pl.kernel,
        mesh=mesh,
        compiler_params=pltpu.CompilerParams(use_tc_tiling_on_sc=False),
        out_type=jax.ShapeDtypeStruct((b_total, D_MODEL), jnp.float32),
        scratch_types=[
            pltpu.VMEM((bpw,), jnp.int32),
            pltpu.VMEM((CHUNK, D_MODEL), jnp.float32),
            pltpu.SemaphoreType.DMA,
        ],
    )
    def body(table_hbm, idx_hbm, out_hbm, idx_v, rows_v, sem):
        wid = lax.axis_index("s") * nc + lax.axis_index("c")
        pltpu.sync_copy(idx_hbm.at[pl.ds(wid * bpw, bpw)], idx_v)
        out_base = wid * bpw

        def chunk_body(g, _):
            idx_sl = idx_v.at[pl.ds(g * CHUNK, CHUNK)]
            pltpu.async_copy(table_hbm.at[idx_sl], rows_v, sem).wait()

            def scale_row(r, _):
                for j in range(D_MODEL // LANES):
                    sl = pl.ds(j * LANES, LANES)
                    rows_v[r, sl] = rows_v[r, sl] * SCALE
                return 0

            lax.fori_loop(0, CHUNK, scale_row, 0)
            pltpu.sync_copy(rows_v, out_hbm.at[pl.ds(out_base + g * CHUNK, CHUNK)])
            return 0

        lax.fori_loop(0, n_chunks, chunk_body, 0)

    return body(table, x1d)


def kernel(x, table):
    b = x.size
    n_workers = 32
    assert b % (n_workers * CHUNK) == 0
    n_chunks = b // (n_workers * CHUNK)
    out = _embed_sc(x.reshape(-1), table, n_chunks=n_chunks, n_workers=n_workers)
    return out.reshape(x.shape + (D_MODEL,))


# speedup vs baseline: 1.0813x; 1.0813x over previous
import functools

import jax
import jax.numpy as jnp
from jax import lax
from jax.experimental import pallas as pl
from jax.experimental.pallas import tpu as pltpu
from jax.experimental.pallas import tpu_sc as plsc

D_MODEL = 64
SCALE = 8.0
CHUNK = 128
GROUP = 5 * CHUNK
LANES = 16
ROW_UNROLL = 4


@functools.partial(jax.jit, static_argnames=("n_groups", "n_workers"))
def _embed_sc(x1d, table, *, n_groups, n_workers):
    b_total = x1d.shape[0]
    bpw = n_groups * GROUP
    info = plsc.get_sparse_core_info()
    nc, ns = info.num_cores, info.num_subcores
    assert nc * ns == n_workers
    mesh = plsc.VectorSubcoreMesh(core_axis_name="c", subcore_axis_name="s")

    @functools.partial(
        pl.kernel,
        mesh=mesh,
        compiler_params=pltpu.CompilerParams(use_tc_tiling_on_sc=False),
        out_type=jax.ShapeDtypeStruct((b_total, D_MODEL), jnp.float32),
        scratch_types=[
            pltpu.VMEM((bpw,), jnp.int32),
            pltpu.VMEM((GROUP, D_MODEL), jnp.float32),
            pltpu.VMEM((GROUP, D_MODEL), jnp.float32),
            pltpu.SemaphoreType.DMA,
            pltpu.SemaphoreType.DMA,
        ],
    )
    def body(table_hbm, idx_hbm, out_hbm, idx_v, rows_a, rows_b, sem_a, sem_b):
        wid = lax.axis_index("s") * nc + lax.axis_index("c")
        pltpu.sync_copy(idx_hbm.at[pl.ds(wid * bpw, bpw)], idx_v)
        out_base = wid * bpw
        bufs = (rows_a, rows_b)
        sems = (sem_a, sem_b)

        def fire(g, b):
            descs = []
            for c in range(GROUP // CHUNK):
                idx_sl = idx_v.at[pl.ds(g * GROUP + c * CHUNK, CHUNK)]
                dst = bufs[b].at[pl.ds(c * CHUNK, CHUNK)]
                descs.append(pltpu.async_copy(table_hbm.at[idx_sl], dst, sems[b]))
            return descs

        def scale(b):
            rows = bufs[b]

            def scale_rows(i, _):
                r0 = i * ROW_UNROLL
                for dr in range(ROW_UNROLL):
                    for j in range(D_MODEL // LANES):
                        sl = pl.ds(j * LANES, LANES)
                        rows[r0 + dr, sl] = rows[r0 + dr, sl] * SCALE
                return 0

            lax.fori_loop(0, GROUP // ROW_UNROLL, scale_rows, 0)

        in_flight = {0: fire(0, 0)}
        for g in range(n_groups):
            b = g & 1
            if g + 1 < n_groups:
                in_flight[g + 1] = fire(g + 1, 1 - b)
            for d in in_flight.pop(g):
                d.wait()
            scale(b)
            pltpu.sync_copy(bufs[b], out_hbm.at[pl.ds(out_base + g * GROUP, GROUP)])

    return body(table, x1d)


def kernel(x, table):
    b = x.size
    n_workers = 32
    assert b % (n_workers * GROUP) == 0
    n_groups = b // (n_workers * GROUP)
    out = _embed_sc(x.reshape(-1), table, n_groups=n_groups, n_workers=n_workers)
    return out.reshape(x.shape + (D_MODEL,))
